# baseline (speedup 1.0000x reference)
import functools

import jax
import jax.numpy as jnp
from jax import lax
from jax.experimental import pallas as pl
from jax.experimental.pallas import tpu as pltpu
from jax.experimental.pallas import tpu_sc as plsc

_N = 10000
_E = 320000
_F = 128
_G = 64
_CLS = 39
_EPS = 1e-6
_MAXN = 1000.0

_NC = 2
_NS = 16
_EPT = _E // (_NC * _NS)
_CH = 128
_NFULL = _EPT // _CH
_TAIL = _EPT - _NFULL * _CH
_BCH = 6
_NBLK = _NFULL // _BCH
_BE = _BCH * _CH
_RPT = 624
_RREM = _N - _NS * _RPT


def _sinh(a):
    return 0.5 * (jnp.exp(a) - jnp.exp(-a))


def _exp_coef(r2, sqrt_c):
    lnorm = jnp.sqrt(jnp.clip(r2 + _EPS, 1e-6, None))
    cut = jnp.minimum(lnorm, 50.0)
    coef = sqrt_c * _sinh(cut / sqrt_c) / lnorm
    m2 = coef * coef * r2
    norm = jnp.sqrt(jnp.clip(m2, 1e-12, None))
    scale = jnp.minimum(1.0, _MAXN / norm)
    sigma = coef * scale
    m2s = jnp.where(m2 == jnp.inf, 0.0, m2 * scale * scale)
    return sigma, m2s


def _log_coef(m2, sqrt_c1, sqrt_c2):
    head = jnp.sqrt(sqrt_c1 * sqrt_c1 + m2)
    z = jnp.clip(head / sqrt_c2 + _EPS, 1.0, None)
    dist = sqrt_c2 * jnp.log(z + jnp.sqrt(jnp.clip(z * z - 1.0, 1e-12,
                                                   None)))
    tmp = jnp.sqrt(jnp.clip(m2, 1e-12, None) + _EPS)
    return dist / tmp


def _pair_sigma(r2, sqrt_c1, sqrt_c2):
    sigma_e, m2 = _exp_coef(r2, sqrt_c1)
    return sigma_e * _log_coef(m2, sqrt_c1, sqrt_c2)


def _xtan_body(x_ref, o_ref):
    t = x_ref[...]
    r2 = jnp.sum(t * t, axis=1, keepdims=True)
    norm = jnp.sqrt(jnp.clip(r2, 1e-12, None))
    scale = jnp.minimum(1.0, _MAXN / norm)
    m2 = r2 * scale * scale
    o_ref[...] = (scale * _log_coef(m2, 1.0, 1.0)) * t


def _xtan(xt):
    b = 2000
    return pl.pallas_call(
        _xtan_body,
        grid=(_N // b,),
        in_specs=[pl.BlockSpec((b, _F), lambda i: (i, 0))],
        out_specs=pl.BlockSpec((b, _F), lambda i: (i, 0)),
        out_shape=jax.ShapeDtypeStruct((_N, _F), jnp.float32),
    )(xt)


def _sc_body(xtan_hbm, src_hbm, dst_hbm, zero_hbm, out_hbm,
             sblk, dblk, rows0, rows1, sidx_t, didx_t, rows_t, acc,
             sem, gsem, ssem):
    c = lax.axis_index("c")
    s = lax.axis_index("s")
    tid = c * _NS + s

    rowbase = s * _RPT
    pltpu.sync_copy(zero_hbm, acc.at[pl.ds(rowbase, _RPT)])

    @pl.when(s == _NS - 1)
    def _():
        pltpu.sync_copy(zero_hbm.at[pl.ds(0, _RREM)],
                        acc.at[pl.ds(_NS * _RPT, _RREM)])

    plsc.subcore_barrier()

    ebase = tid * _EPT
    rows = (rows0, rows1)

    def blk_body(blk, carry):
        base = ebase + blk * _BE
        pltpu.sync_copy(src_hbm.at[pl.ds(base, _BE)], sblk)
        pltpu.sync_copy(dst_hbm.at[pl.ds(base, _BE)], dblk)
        g = [None] * _BCH
        s = [None] * _BCH
        g[0] = pltpu.async_copy(xtan_hbm.at[sblk.at[pl.ds(0, _CH)]],
                                rows0, gsem)
        for k in range(1, _BCH):
            cur = rows[(k - 1) % 2]
            nxt = rows[k % 2]
            if k >= 2:
                s[k - 2].wait()
            g[k] = pltpu.async_copy(
                xtan_hbm.at[sblk.at[pl.ds(k * _CH, _CH)]], nxt, gsem)
            g[k - 1].wait()
            s[k - 1] = pltpu.async_copy(
                cur, acc.at[dblk.at[pl.ds((k - 1) * _CH, _CH)]],
                ssem, add=True)
        last = rows[(_BCH - 1) % 2]
        g[_BCH - 1].wait()
        s[_BCH - 1] = pltpu.async_copy(
            last, acc.at[dblk.at[pl.ds((_BCH - 1) * _CH, _CH)]],
            ssem, add=True)
        s[_BCH - 2].wait()
        s[_BCH - 1].wait()
        return carry

    lax.fori_loop(0, _NBLK, blk_body, 0)

    start = ebase + _NFULL * _CH
    pltpu.sync_copy(src_hbm.at[pl.ds(start, _TAIL)], sidx_t)
    pltpu.sync_copy(dst_hbm.at[pl.ds(start, _TAIL)], didx_t)
    pltpu.async_copy(xtan_hbm.at[sidx_t], rows_t, sem).wait()
    pltpu.sync_copy(rows_t, acc.at[didx_t], add=True)

    plsc.subcore_barrier()

    pltpu.sync_copy(acc.at[pl.ds(rowbase, _RPT)],
                    out_hbm.at[pl.ds(c * _N + rowbase, _RPT)])

    @pl.when(s == _NS - 1)
    def _():
        pltpu.sync_copy(acc.at[pl.ds(_NS * _RPT, _RREM)],
                        out_hbm.at[pl.ds(c * _N + _NS * _RPT, _RREM)])


def _sc_scatter(x_tan, src, dst, zero_rows):
    mesh = plsc.VectorSubcoreMesh(core_axis_name="c", subcore_axis_name="s")
    f = pl.kernel(
        _sc_body,
        out_type=jax.ShapeDtypeStruct((_NC * _N, _F), jnp.float32),
        mesh=mesh,
        scratch_types=[
            pltpu.VMEM((_BE,), jnp.int32),
            pltpu.VMEM((_BE,), jnp.int32),
            pltpu.VMEM((_CH, _F), jnp.float32),
            pltpu.VMEM((_CH, _F), jnp.float32),
            pltpu.VMEM((_TAIL,), jnp.int32),
            pltpu.VMEM((_TAIL,), jnp.int32),
            pltpu.VMEM((_TAIL, _F), jnp.float32),
            pltpu.VMEM_SHARED((_N, _F), jnp.float32),
            pltpu.SemaphoreType.DMA,
            pltpu.SemaphoreType.DMA,
            pltpu.SemaphoreType.DMA,
        ],
    )
    return f(x_tan, src, dst, zero_rows)


_B2 = 2000
_NB2 = _N // _B2


_FH = _F - 1


def _tail_body(xt_ref, p0_ref, p1_ref, b_ref,
               w0_ref, b0_ref, w1_ref, b1_ref, wc_ref, bc_ref,
               olog_ref, oprob_ref, acc_ref):
    i = pl.program_id(0)
    dn = (((1,), (1,)), ((), ()))

    htan = xt_ref[...] + p0_ref[...] + p1_ref[...]
    s1 = _pair_sigma(jnp.sum(htan * htan, axis=1, keepdims=True), 1.0, 2.0)
    mx = s1 * lax.dot_general(htan, w0_ref[...], dn,
                              preferred_element_type=jnp.float32) \
        + b0_ref[...]
    s2 = _pair_sigma(jnp.sum(mx * mx, axis=1, keepdims=True), 2.0, 2.0)
    a = jax.nn.relu(mx)
    ra = jnp.sum(a * a, axis=1, keepdims=True)
    s3 = _pair_sigma(s2 * s2 * ra, 2.0, 2.0)
    mx = (s2 * s3) * lax.dot_general(a, w1_ref[...], dn,
                                     preferred_element_type=jnp.float32) \
        + b1_ref[...]
    s4 = _pair_sigma(jnp.sum(mx * mx, axis=1, keepdims=True), 2.0, 2.0)
    a = jax.nn.relu(mx)
    ra = jnp.sum(a * a, axis=1, keepdims=True)
    s5 = _pair_sigma(s4 * s4 * ra, 1.0, 1.0)
    tt = (s4 * s5) * a

    bvals = b_ref[...].reshape(1, _B2)
    gid = lax.broadcasted_iota(jnp.int32, (_G, _B2), 0)
    oh = jnp.where(gid == bvals, 1.0, 0.0)
    pp = jnp.dot(oh, tt, preferred_element_type=jnp.float32)

    @pl.when(i == 0)
    def _():
        acc_ref[...] = pp

    @pl.when(i > 0)
    def _():
        acc_ref[...] = acc_ref[...] + pp

    @pl.when(i == _NB2 - 1)
    def _():
        hp = acc_ref[...]
        s6 = _pair_sigma(jnp.sum(hp * hp, axis=1, keepdims=True), 1.0, 1.0)
        mxc = s6 * lax.dot_general(hp, wc_ref[...], dn,
                                   preferred_element_type=jnp.float32) \
            + bc_ref[...]
        s7 = _pair_sigma(jnp.sum(mxc * mxc, axis=1, keepdims=True), 1.0, 1.0)
        lt = s7 * mxc
        olog_ref[...] = lt
        m = jnp.maximum(jnp.max(lt, axis=1, keepdims=True), 0.0)
        e = jnp.exp(lt - m)
        denom = jnp.sum(e, axis=1, keepdims=True) + jnp.exp(-m)
        st = e / denom
        se, _ = _exp_coef(jnp.sum(st * st, axis=1, keepdims=True), 1.0)
        oprob_ref[...] = se * st


def _tail_call(x_tan, p0, p1, batch3, w0, b0, w1, b1, wc, bc):
    blk = lambda i: (i, 0)
    fixed = lambda i: (0, 0)
    return pl.pallas_call(
        _tail_body,
        grid=(_NB2,),
        in_specs=[
            pl.BlockSpec((_B2, _F), blk),
            pl.BlockSpec((_B2, _F), blk),
            pl.BlockSpec((_B2, _F), blk),
            pl.BlockSpec((1, 1, _B2), lambda i: (i, 0, 0)),
            pl.BlockSpec((_FH, _F), fixed),
            pl.BlockSpec((1, _FH), fixed),
            pl.BlockSpec((_FH, _FH), fixed),
            pl.BlockSpec((1, _FH), fixed),
            pl.BlockSpec((_CLS, _FH), fixed),
            pl.BlockSpec((1, _CLS), fixed),
        ],
        out_specs=[
            pl.BlockSpec((_G, _CLS), fixed),
            pl.BlockSpec((_G, _CLS), fixed),
        ],
        out_shape=[
            jax.ShapeDtypeStruct((_G, _CLS), jnp.float32),
            jax.ShapeDtypeStruct((_G, _CLS), jnp.float32),
        ],
        scratch_shapes=[pltpu.VMEM((_G, _FH), jnp.float32)],
    )(x_tan, p0, p1, batch3, w0, b0, w1, b1, wc, bc)


def kernel(x, edge_index, batch, W0, b0, W1, b1, Wc, bc):
    xt = x[:, 1:]
    x_tan = _xtan(xt)

    src = edge_index[0]
    dst = edge_index[1]
    zero_rows = jnp.zeros((_RPT, _F), jnp.float32)
    parts = _sc_scatter(x_tan, src, dst, zero_rows)
    p0 = parts[:_N]
    p1 = parts[_N:]

    batch3 = batch.reshape(_NB2, 1, _B2)

    olog, oprob = _tail_call(x_tan, p0, p1, batch3,
                             W0, b0.reshape(1, _FH),
                             W1, b1.reshape(1, _FH),
                             Wc, bc.reshape(1, _CLS))
    return olog, oprob


# speedup vs baseline: 1.0298x; 1.0298x over previous
import functools

import jax
import jax.numpy as jnp
from jax import lax
from jax.experimental import pallas as pl
from jax.experimental.pallas import tpu as pltpu
from jax.experimental.pallas import tpu_sc as plsc

_N = 10000
_E = 320000
_F = 128
_G = 64
_CLS = 39
_EPS = 1e-6
_MAXN = 1000.0

_NC = 2
_NS = 16
_EPT = _E // (_NC * _NS)
_CH = 128
_NFULL = _EPT // _CH
_TAIL = _EPT - _NFULL * _CH
_BCH = 6
_NBLK = _NFULL // _BCH
_BE = _BCH * _CH
_RPT = 624
_RREM = _N - _NS * _RPT


def _sinh(a):
    return 0.5 * (jnp.exp(a) - jnp.exp(-a))


def _exp_coef(r2, sqrt_c):
    lnorm = jnp.sqrt(jnp.clip(r2 + _EPS, 1e-6, None))
    cut = jnp.minimum(lnorm, 50.0)
    coef = sqrt_c * _sinh(cut / sqrt_c) / lnorm
    m2 = coef * coef * r2
    norm = jnp.sqrt(jnp.clip(m2, 1e-12, None))
    scale = jnp.minimum(1.0, _MAXN / norm)
    sigma = coef * scale
    m2s = jnp.where(m2 == jnp.inf, 0.0, m2 * scale * scale)
    return sigma, m2s


def _log_coef(m2, sqrt_c1, sqrt_c2):
    head = jnp.sqrt(sqrt_c1 * sqrt_c1 + m2)
    z = jnp.clip(head / sqrt_c2 + _EPS, 1.0, None)
    dist = sqrt_c2 * jnp.log(z + jnp.sqrt(jnp.clip(z * z - 1.0, 1e-12,
                                                   None)))
    tmp = jnp.sqrt(jnp.clip(m2, 1e-12, None) + _EPS)
    return dist / tmp


def _pair_sigma(r2, sqrt_c1, sqrt_c2):
    sigma_e, m2 = _exp_coef(r2, sqrt_c1)
    return sigma_e * _log_coef(m2, sqrt_c1, sqrt_c2)


def _xtan_body(x_ref, o_ref):
    t = x_ref[...]
    r2 = jnp.sum(t * t, axis=1, keepdims=True)
    norm = jnp.sqrt(jnp.clip(r2, 1e-12, None))
    scale = jnp.minimum(1.0, _MAXN / norm)
    m2 = r2 * scale * scale
    o_ref[...] = (scale * _log_coef(m2, 1.0, 1.0)) * t


def _xtan(xt):
    b = 2000
    return pl.pallas_call(
        _xtan_body,
        grid=(_N // b,),
        in_specs=[pl.BlockSpec((b, _F), lambda i: (i, 0))],
        out_specs=pl.BlockSpec((b, _F), lambda i: (i, 0)),
        out_shape=jax.ShapeDtypeStruct((_N, _F), jnp.float32),
    )(xt)


def _sc_body(xtan_hbm, src_hbm, dst_hbm, zero_hbm, out_hbm,
             sblk, dblk, rows0, rows1, sidx_t, didx_t, rows_t, acc,
             sem, gsem, ssem):
    c = lax.axis_index("c")
    s = lax.axis_index("s")
    tid = c * _NS + s

    rowbase = s * _RPT
    pltpu.sync_copy(zero_hbm, acc.at[pl.ds(rowbase, _RPT)])

    @pl.when(s == _NS - 1)
    def _():
        pltpu.sync_copy(zero_hbm.at[pl.ds(0, _RREM)],
                        acc.at[pl.ds(_NS * _RPT, _RREM)])

    plsc.subcore_barrier()

    ebase = tid * _EPT
    rows = (rows0, rows1)

    def blk_body(blk, carry):
        base = ebase + blk * _BE
        pltpu.sync_copy(src_hbm.at[pl.ds(base, _BE)], sblk)
        pltpu.sync_copy(dst_hbm.at[pl.ds(base, _BE)], dblk)
        g = [None] * _BCH
        s = [None] * _BCH
        g[0] = pltpu.async_copy(xtan_hbm.at[sblk.at[pl.ds(0, _CH)]],
                                rows0, gsem)
        for k in range(1, _BCH):
            cur = rows[(k - 1) % 2]
            nxt = rows[k % 2]
            if k >= 2:
                s[k - 2].wait()
            g[k] = pltpu.async_copy(
                xtan_hbm.at[sblk.at[pl.ds(k * _CH, _CH)]], nxt, gsem)
            g[k - 1].wait()
            s[k - 1] = pltpu.async_copy(
                cur, acc.at[dblk.at[pl.ds((k - 1) * _CH, _CH)]],
                ssem, add=True)
        last = rows[(_BCH - 1) % 2]
        g[_BCH - 1].wait()
        s[_BCH - 1] = pltpu.async_copy(
            last, acc.at[dblk.at[pl.ds((_BCH - 1) * _CH, _CH)]],
            ssem, add=True)
        s[_BCH - 2].wait()
        s[_BCH - 1].wait()
        return carry

    lax.fori_loop(0, _NBLK, blk_body, 0)

    start = ebase + _NFULL * _CH
    pltpu.sync_copy(src_hbm.at[pl.ds(start, _TAIL)], sidx_t)
    pltpu.sync_copy(dst_hbm.at[pl.ds(start, _TAIL)], didx_t)
    pltpu.async_copy(xtan_hbm.at[sidx_t], rows_t, sem).wait()
    pltpu.sync_copy(rows_t, acc.at[didx_t], add=True)

    plsc.subcore_barrier()

    pltpu.sync_copy(acc.at[pl.ds(rowbase, _RPT)],
                    out_hbm.at[pl.ds(c * _N + rowbase, _RPT)])

    @pl.when(s == _NS - 1)
    def _():
        pltpu.sync_copy(acc.at[pl.ds(_NS * _RPT, _RREM)],
                        out_hbm.at[pl.ds(c * _N + _NS * _RPT, _RREM)])


def _sc_scatter(x_tan, src, dst, zero_rows):
    mesh = plsc.VectorSubcoreMesh(core_axis_name="c", subcore_axis_name="s")
    f = pl.kernel(
        _sc_body,
        out_type=jax.ShapeDtypeStruct((_NC * _N, _F), jnp.float32),
        mesh=mesh,
        scratch_types=[
            pltpu.VMEM((_BE,), jnp.int32),
            pltpu.VMEM((_BE,), jnp.int32),
            pltpu.VMEM((_CH, _F), jnp.float32),
            pltpu.VMEM((_CH, _F), jnp.float32),
            pltpu.VMEM((_TAIL,), jnp.int32),
            pltpu.VMEM((_TAIL,), jnp.int32),
            pltpu.VMEM((_TAIL, _F), jnp.float32),
            pltpu.VMEM_SHARED((_N, _F), jnp.float32),
            pltpu.SemaphoreType.DMA,
            pltpu.SemaphoreType.DMA,
            pltpu.SemaphoreType.DMA,
        ],
    )
    return f(x_tan, src, dst, zero_rows)


_B2 = 2000
_NB2 = _N // _B2


_FH = _F - 1


def _tail_body(xt_ref, p0_ref, p1_ref, b_ref,
               w0_ref, b0_ref, w1_ref, b1_ref, wc_ref, bc_ref,
               olog_ref, oprob_ref, acc_ref):
    i = pl.program_id(0)
    dn = (((1,), (1,)), ((), ()))

    htan = xt_ref[...] + p0_ref[...] + p1_ref[...]
    s1 = _pair_sigma(jnp.sum(htan * htan, axis=1, keepdims=True), 1.0, 2.0)
    mx = s1 * lax.dot_general(htan, w0_ref[...], dn,
                              preferred_element_type=jnp.float32) \
        + b0_ref[...]
    s2 = _pair_sigma(jnp.sum(mx * mx, axis=1, keepdims=True), 2.0, 2.0)
    a = jax.nn.relu(mx)
    ra = jnp.sum(a * a, axis=1, keepdims=True)
    s3 = _pair_sigma(s2 * s2 * ra, 2.0, 2.0)
    mx = (s2 * s3) * lax.dot_general(a, w1_ref[...], dn,
                                     preferred_element_type=jnp.float32) \
        + b1_ref[...]
    s4 = _pair_sigma(jnp.sum(mx * mx, axis=1, keepdims=True), 2.0, 2.0)
    a = jax.nn.relu(mx)
    ra = jnp.sum(a * a, axis=1, keepdims=True)
    s5 = _pair_sigma(s4 * s4 * ra, 1.0, 1.0)
    tt = (s4 * s5) * a

    bvals = b_ref[...].reshape(1, _B2)
    gid = lax.broadcasted_iota(jnp.int32, (_G, _B2), 0)
    oh = jnp.where(gid == bvals, 1.0, 0.0)
    pp = jnp.dot(oh, tt, preferred_element_type=jnp.float32)

    @pl.when(i == 0)
    def _():
        acc_ref[...] = pp

    @pl.when(i > 0)
    def _():
        acc_ref[...] = acc_ref[...] + pp

    @pl.when(i == _NB2 - 1)
    def _():
        hp = acc_ref[...]
        s6 = _pair_sigma(jnp.sum(hp * hp, axis=1, keepdims=True), 1.0, 1.0)
        mxc = s6 * lax.dot_general(hp, wc_ref[...], dn,
                                   preferred_element_type=jnp.float32) \
            + bc_ref[...]
        s7 = _pair_sigma(jnp.sum(mxc * mxc, axis=1, keepdims=True), 1.0, 1.0)
        lt = s7 * mxc
        olog_ref[...] = lt
        m = jnp.maximum(jnp.max(lt, axis=1, keepdims=True), 0.0)
        e = jnp.exp(lt - m)
        denom = jnp.sum(e, axis=1, keepdims=True) + jnp.exp(-m)
        st = e / denom
        se, _ = _exp_coef(jnp.sum(st * st, axis=1, keepdims=True), 1.0)
        oprob_ref[...] = se * st


def _tail_call(x_tan, p0, p1, batch3, w0, b0, w1, b1, wc, bc):
    blk = lambda i: (i, 0)
    fixed = lambda i: (0, 0)
    return pl.pallas_call(
        _tail_body,
        grid=(_NB2,),
        in_specs=[
            pl.BlockSpec((_B2, _F), blk),
            pl.BlockSpec((_B2, _F), blk),
            pl.BlockSpec((_B2, _F), lambda i: (i + _NB2, 0)),
            pl.BlockSpec((1, 1, _B2), lambda i: (i, 0, 0)),
            pl.BlockSpec((_FH, _F), fixed),
            pl.BlockSpec((1, _FH), fixed),
            pl.BlockSpec((_FH, _FH), fixed),
            pl.BlockSpec((1, _FH), fixed),
            pl.BlockSpec((_CLS, _FH), fixed),
            pl.BlockSpec((1, _CLS), fixed),
        ],
        out_specs=[
            pl.BlockSpec((_G, _CLS), fixed),
            pl.BlockSpec((_G, _CLS), fixed),
        ],
        out_shape=[
            jax.ShapeDtypeStruct((_G, _CLS), jnp.float32),
            jax.ShapeDtypeStruct((_G, _CLS), jnp.float32),
        ],
        scratch_shapes=[pltpu.VMEM((_G, _FH), jnp.float32)],
    )(x_tan, p0, p1, batch3, w0, b0, w1, b1, wc, bc)


def kernel(x, edge_index, batch, W0, b0, W1, b1, Wc, bc):
    xt = x[:, 1:]
    x_tan = _xtan(xt)

    src = edge_index[0]
    dst = edge_index[1]
    zero_rows = jnp.zeros((_RPT, _F), jnp.float32)
    parts = _sc_scatter(x_tan, src, dst, zero_rows)

    batch3 = batch.reshape(_NB2, 1, _B2)

    olog, oprob = _tail_call(x_tan, parts, parts, batch3,
                             W0, b0.reshape(1, _FH),
                             W1, b1.reshape(1, _FH),
                             Wc, bc.reshape(1, _CLS))
    return olog, oprob


# speedup vs baseline: 1.0687x; 1.0378x over previous
import functools

import jax
import jax.numpy as jnp
from jax import lax
from jax.experimental import pallas as pl
from jax.experimental.pallas import tpu as pltpu
from jax.experimental.pallas import tpu_sc as plsc

_N = 10000
_E = 320000
_F = 128
_G = 64
_CLS = 39
_EPS = 1e-6
_MAXN = 1000.0

_NC = 2
_NS = 16
_EPT = _E // (_NC * _NS)
_CH = 128
_NFULL = _EPT // _CH
_TAIL = _EPT - _NFULL * _CH
_BCH = 13
_NBLK = _NFULL // _BCH
_BE = _BCH * _CH
_RPT = 624
_RREM = _N - _NS * _RPT


def _sinh(a):
    return 0.5 * (jnp.exp(a) - jnp.exp(-a))


def _exp_coef(r2, sqrt_c):
    lnorm = jnp.sqrt(jnp.clip(r2 + _EPS, 1e-6, None))
    cut = jnp.minimum(lnorm, 50.0)
    coef = sqrt_c * _sinh(cut / sqrt_c) / lnorm
    m2 = coef * coef * r2
    norm = jnp.sqrt(jnp.clip(m2, 1e-12, None))
    scale = jnp.minimum(1.0, _MAXN / norm)
    sigma = coef * scale
    m2s = jnp.where(m2 == jnp.inf, 0.0, m2 * scale * scale)
    return sigma, m2s


def _log_coef(m2, sqrt_c1, sqrt_c2):
    head = jnp.sqrt(sqrt_c1 * sqrt_c1 + m2)
    z = jnp.clip(head / sqrt_c2 + _EPS, 1.0, None)
    dist = sqrt_c2 * jnp.log(z + jnp.sqrt(jnp.clip(z * z - 1.0, 1e-12,
                                                   None)))
    tmp = jnp.sqrt(jnp.clip(m2, 1e-12, None) + _EPS)
    return dist / tmp


def _pair_sigma(r2, sqrt_c1, sqrt_c2):
    sigma_e, m2 = _exp_coef(r2, sqrt_c1)
    return sigma_e * _log_coef(m2, sqrt_c1, sqrt_c2)


def _xtan_body(x_ref, o_ref):
    t = x_ref[...]
    r2 = jnp.sum(t * t, axis=1, keepdims=True)
    norm = jnp.sqrt(jnp.clip(r2, 1e-12, None))
    scale = jnp.minimum(1.0, _MAXN / norm)
    m2 = r2 * scale * scale
    o_ref[...] = (scale * _log_coef(m2, 1.0, 1.0)) * t


def _xtan(xt):
    b = 2000
    return pl.pallas_call(
        _xtan_body,
        grid=(_N // b,),
        in_specs=[pl.BlockSpec((b, _F), lambda i: (i, 0))],
        out_specs=pl.BlockSpec((b, _F), lambda i: (i, 0)),
        out_shape=jax.ShapeDtypeStruct((_N, _F), jnp.float32),
    )(xt)


def _sc_body(xtan_hbm, src_hbm, dst_hbm, zero_hbm, out_hbm,
             sblk, dblk, rows0, rows1, sidx_t, didx_t, rows_t, acc,
             sem, gsem, ssem):
    c = lax.axis_index("c")
    s = lax.axis_index("s")
    tid = c * _NS + s

    rowbase = s * _RPT
    pltpu.sync_copy(zero_hbm, acc.at[pl.ds(rowbase, _RPT)])

    @pl.when(s == _NS - 1)
    def _():
        pltpu.sync_copy(zero_hbm.at[pl.ds(0, _RREM)],
                        acc.at[pl.ds(_NS * _RPT, _RREM)])

    plsc.subcore_barrier()

    ebase = tid * _EPT
    rows = (rows0, rows1)

    def blk_body(blk, carry):
        base = ebase + blk * _BE
        pltpu.sync_copy(src_hbm.at[pl.ds(base, _BE)], sblk)
        pltpu.sync_copy(dst_hbm.at[pl.ds(base, _BE)], dblk)
        g = [None] * _BCH
        s = [None] * _BCH
        g[0] = pltpu.async_copy(xtan_hbm.at[sblk.at[pl.ds(0, _CH)]],
                                rows0, gsem)
        for k in range(1, _BCH):
            cur = rows[(k - 1) % 2]
            nxt = rows[k % 2]
            if k >= 2:
                s[k - 2].wait()
            g[k] = pltpu.async_copy(
                xtan_hbm.at[sblk.at[pl.ds(k * _CH, _CH)]], nxt, gsem)
            g[k - 1].wait()
            s[k - 1] = pltpu.async_copy(
                cur, acc.at[dblk.at[pl.ds((k - 1) * _CH, _CH)]],
                ssem, add=True)
        last = rows[(_BCH - 1) % 2]
        g[_BCH - 1].wait()
        s[_BCH - 1] = pltpu.async_copy(
            last, acc.at[dblk.at[pl.ds((_BCH - 1) * _CH, _CH)]],
            ssem, add=True)
        s[_BCH - 2].wait()
        s[_BCH - 1].wait()
        return carry

    lax.fori_loop(0, _NBLK, blk_body, 0)

    start = ebase + _NFULL * _CH
    pltpu.sync_copy(src_hbm.at[pl.ds(start, _TAIL)], sidx_t)
    pltpu.sync_copy(dst_hbm.at[pl.ds(start, _TAIL)], didx_t)
    pltpu.async_copy(xtan_hbm.at[sidx_t], rows_t, sem).wait()
    pltpu.sync_copy(rows_t, acc.at[didx_t], add=True)

    plsc.subcore_barrier()

    pltpu.sync_copy(acc.at[pl.ds(rowbase, _RPT)],
                    out_hbm.at[pl.ds(c * _N + rowbase, _RPT)])

    @pl.when(s == _NS - 1)
    def _():
        pltpu.sync_copy(acc.at[pl.ds(_NS * _RPT, _RREM)],
                        out_hbm.at[pl.ds(c * _N + _NS * _RPT, _RREM)])


def _sc_scatter(x_tan, src, dst, zero_rows):
    mesh = plsc.VectorSubcoreMesh(core_axis_name="c", subcore_axis_name="s")
    f = pl.kernel(
        _sc_body,
        out_type=jax.ShapeDtypeStruct((_NC * _N, _F), jnp.float32),
        mesh=mesh,
        scratch_types=[
            pltpu.VMEM((_BE,), jnp.int32),
            pltpu.VMEM((_BE,), jnp.int32),
            pltpu.VMEM((_CH, _F), jnp.float32),
            pltpu.VMEM((_CH, _F), jnp.float32),
            pltpu.VMEM((_TAIL,), jnp.int32),
            pltpu.VMEM((_TAIL,), jnp.int32),
            pltpu.VMEM((_TAIL, _F), jnp.float32),
            pltpu.VMEM_SHARED((_N, _F), jnp.float32),
            pltpu.SemaphoreType.DMA,
            pltpu.SemaphoreType.DMA,
            pltpu.SemaphoreType.DMA,
        ],
    )
    return f(x_tan, src, dst, zero_rows)


_B2 = 2000
_NB2 = _N // _B2


_FH = _F - 1


def _tail_body(xt_ref, p0_ref, p1_ref, b_ref,
               w0_ref, b0_ref, w1_ref, b1_ref, wc_ref, bc_ref,
               olog_ref, oprob_ref, acc_ref):
    i = pl.program_id(0)
    dn = (((1,), (1,)), ((), ()))

    htan = xt_ref[...] + p0_ref[...] + p1_ref[...]
    s1 = _pair_sigma(jnp.sum(htan * htan, axis=1, keepdims=True), 1.0, 2.0)
    mx = s1 * lax.dot_general(htan, w0_ref[...], dn,
                              preferred_element_type=jnp.float32) \
        + b0_ref[...]
    s2 = _pair_sigma(jnp.sum(mx * mx, axis=1, keepdims=True), 2.0, 2.0)
    a = jax.nn.relu(mx)
    ra = jnp.sum(a * a, axis=1, keepdims=True)
    s3 = _pair_sigma(s2 * s2 * ra, 2.0, 2.0)
    mx = (s2 * s3) * lax.dot_general(a, w1_ref[...], dn,
                                     preferred_element_type=jnp.float32) \
        + b1_ref[...]
    s4 = _pair_sigma(jnp.sum(mx * mx, axis=1, keepdims=True), 2.0, 2.0)
    a = jax.nn.relu(mx)
    ra = jnp.sum(a * a, axis=1, keepdims=True)
    s5 = _pair_sigma(s4 * s4 * ra, 1.0, 1.0)
    tt = (s4 * s5) * a

    bvals = b_ref[...].reshape(1, _B2)
    gid = lax.broadcasted_iota(jnp.int32, (_G, _B2), 0)
    oh = jnp.where(gid == bvals, 1.0, 0.0)
    pp = jnp.dot(oh, tt, preferred_element_type=jnp.float32)

    @pl.when(i == 0)
    def _():
        acc_ref[...] = pp

    @pl.when(i > 0)
    def _():
        acc_ref[...] = acc_ref[...] + pp

    @pl.when(i == _NB2 - 1)
    def _():
        hp = acc_ref[...]
        s6 = _pair_sigma(jnp.sum(hp * hp, axis=1, keepdims=True), 1.0, 1.0)
        mxc = s6 * lax.dot_general(hp, wc_ref[...], dn,
                                   preferred_element_type=jnp.float32) \
            + bc_ref[...]
        s7 = _pair_sigma(jnp.sum(mxc * mxc, axis=1, keepdims=True), 1.0, 1.0)
        lt = s7 * mxc
        olog_ref[...] = lt
        m = jnp.maximum(jnp.max(lt, axis=1, keepdims=True), 0.0)
        e = jnp.exp(lt - m)
        denom = jnp.sum(e, axis=1, keepdims=True) + jnp.exp(-m)
        st = e / denom
        se, _ = _exp_coef(jnp.sum(st * st, axis=1, keepdims=True), 1.0)
        oprob_ref[...] = se * st


def _tail_call(x_tan, p0, p1, batch3, w0, b0, w1, b1, wc, bc):
    blk = lambda i: (i, 0)
    fixed = lambda i: (0, 0)
    return pl.pallas_call(
        _tail_body,
        grid=(_NB2,),
        in_specs=[
            pl.BlockSpec((_B2, _F), blk),
            pl.BlockSpec((_B2, _F), blk),
            pl.BlockSpec((_B2, _F), lambda i: (i + _NB2, 0)),
            pl.BlockSpec((1, 1, _B2), lambda i: (i, 0, 0)),
            pl.BlockSpec((_FH, _F), fixed),
            pl.BlockSpec((1, _FH), fixed),
            pl.BlockSpec((_FH, _FH), fixed),
            pl.BlockSpec((1, _FH), fixed),
            pl.BlockSpec((_CLS, _FH), fixed),
            pl.BlockSpec((1, _CLS), fixed),
        ],
        out_specs=[
            pl.BlockSpec((_G, _CLS), fixed),
            pl.BlockSpec((_G, _CLS), fixed),
        ],
        out_shape=[
            jax.ShapeDtypeStruct((_G, _CLS), jnp.float32),
            jax.ShapeDtypeStruct((_G, _CLS), jnp.float32),
        ],
        scratch_shapes=[pltpu.VMEM((_G, _FH), jnp.float32)],
    )(x_tan, p0, p1, batch3, w0, b0, w1, b1, wc, bc)


def kernel(x, edge_index, batch, W0, b0, W1, b1, Wc, bc):
    xt = x[:, 1:]
    x_tan = _xtan(xt)

    src = edge_index[0]
    dst = edge_index[1]
    zero_rows = jnp.zeros((_RPT, _F), jnp.float32)
    parts = _sc_scatter(x_tan, src, dst, zero_rows)

    batch3 = batch.reshape(_NB2, 1, _B2)

    olog, oprob = _tail_call(x_tan, parts, parts, batch3,
                             W0, b0.reshape(1, _FH),
                             W1, b1.reshape(1, _FH),
                             Wc, bc.reshape(1, _CLS))
    return olog, oprob


# speedup vs baseline: 1.0793x; 1.0099x over previous
import functools

import jax
import jax.numpy as jnp
from jax import lax
from jax.experimental import pallas as pl
from jax.experimental.pallas import tpu as pltpu
from jax.experimental.pallas import tpu_sc as plsc

_N = 10000
_E = 320000
_F = 128
_G = 64
_CLS = 39
_EPS = 1e-6
_MAXN = 1000.0

_NC = 2
_NS = 16
_EPT = _E // (_NC * _NS)
_CH = 128
_NFULL = _EPT // _CH
_TAIL = _EPT - _NFULL * _CH
_BCH = 6
_NBLK = _NFULL // _BCH
_BE = _BCH * _CH
_NPAIR = (_NBLK - 1) // 2
_RPT = 624
_RREM = _N - _NS * _RPT


def _sinh(a):
    return 0.5 * (jnp.exp(a) - jnp.exp(-a))


def _exp_coef(r2, sqrt_c):
    lnorm = jnp.sqrt(jnp.clip(r2 + _EPS, 1e-6, None))
    cut = jnp.minimum(lnorm, 50.0)
    coef = sqrt_c * _sinh(cut / sqrt_c) / lnorm
    m2 = coef * coef * r2
    norm = jnp.sqrt(jnp.clip(m2, 1e-12, None))
    scale = jnp.minimum(1.0, _MAXN / norm)
    sigma = coef * scale
    m2s = jnp.where(m2 == jnp.inf, 0.0, m2 * scale * scale)
    return sigma, m2s


def _log_coef(m2, sqrt_c1, sqrt_c2):
    head = jnp.sqrt(sqrt_c1 * sqrt_c1 + m2)
    z = jnp.clip(head / sqrt_c2 + _EPS, 1.0, None)
    dist = sqrt_c2 * jnp.log(z + jnp.sqrt(jnp.clip(z * z - 1.0, 1e-12,
                                                   None)))
    tmp = jnp.sqrt(jnp.clip(m2, 1e-12, None) + _EPS)
    return dist / tmp


def _pair_sigma(r2, sqrt_c1, sqrt_c2):
    sigma_e, m2 = _exp_coef(r2, sqrt_c1)
    return sigma_e * _log_coef(m2, sqrt_c1, sqrt_c2)


def _xtan_body(x_ref, o_ref):
    t = x_ref[...]
    r2 = jnp.sum(t * t, axis=1, keepdims=True)
    norm = jnp.sqrt(jnp.clip(r2, 1e-12, None))
    scale = jnp.minimum(1.0, _MAXN / norm)
    m2 = r2 * scale * scale
    o_ref[...] = (scale * _log_coef(m2, 1.0, 1.0)) * t


def _xtan(xt):
    b = 2000
    return pl.pallas_call(
        _xtan_body,
        grid=(_N // b,),
        in_specs=[pl.BlockSpec((b, _F), lambda i: (i, 0))],
        out_specs=pl.BlockSpec((b, _F), lambda i: (i, 0)),
        out_shape=jax.ShapeDtypeStruct((_N, _F), jnp.float32),
    )(xt)


def _sc_body(xtan_hbm, src_hbm, dst_hbm, zero_hbm, out_hbm,
             sblk, dblk, sblk2, dblk2, rows0, rows1, sidx_t, didx_t,
             rows_t, acc, sem, gsem, ssem, isem):
    c = lax.axis_index("c")
    s = lax.axis_index("s")
    tid = c * _NS + s

    rowbase = s * _RPT
    pltpu.sync_copy(zero_hbm, acc.at[pl.ds(rowbase, _RPT)])

    @pl.when(s == _NS - 1)
    def _():
        pltpu.sync_copy(zero_hbm.at[pl.ds(0, _RREM)],
                        acc.at[pl.ds(_NS * _RPT, _RREM)])

    plsc.subcore_barrier()

    ebase = tid * _EPT
    rows = (rows0, rows1)

    def run_block(sb, db):
        g = [None] * _BCH
        s = [None] * _BCH
        g[0] = pltpu.async_copy(xtan_hbm.at[sb.at[pl.ds(0, _CH)]],
                                rows0, gsem)
        for k in range(1, _BCH):
            cur = rows[(k - 1) % 2]
            nxt = rows[k % 2]
            if k >= 2:
                s[k - 2].wait()
            g[k] = pltpu.async_copy(
                xtan_hbm.at[sb.at[pl.ds(k * _CH, _CH)]], nxt, gsem)
            g[k - 1].wait()
            s[k - 1] = pltpu.async_copy(
                cur, acc.at[db.at[pl.ds((k - 1) * _CH, _CH)]],
                ssem, add=True)
        last = rows[(_BCH - 1) % 2]
        g[_BCH - 1].wait()
        s[_BCH - 1] = pltpu.async_copy(
            last, acc.at[db.at[pl.ds((_BCH - 1) * _CH, _CH)]],
            ssem, add=True)
        s[_BCH - 2].wait()
        s[_BCH - 1].wait()

    def pf(blk, sb, db):
        base = ebase + jnp.minimum(blk, _NBLK - 1) * _BE
        pltpu.async_copy(src_hbm.at[pl.ds(base, _BE)], sb, isem)
        pltpu.async_copy(dst_hbm.at[pl.ds(base, _BE)], db, isem)

    def pf_wait(sb, db):
        pltpu.make_async_copy(src_hbm.at[pl.ds(ebase, _BE)], sb, isem).wait()
        pltpu.make_async_copy(dst_hbm.at[pl.ds(ebase, _BE)], db, isem).wait()

    pltpu.sync_copy(src_hbm.at[pl.ds(ebase, _BE)], sblk)
    pltpu.sync_copy(dst_hbm.at[pl.ds(ebase, _BE)], dblk)
    pf(1, sblk2, dblk2)
    run_block(sblk, dblk)

    def pair_body(i, carry):
        pf_wait(sblk2, dblk2)
        pf(2 * i + 2, sblk, dblk)
        run_block(sblk2, dblk2)
        pf_wait(sblk, dblk)
        pf(2 * i + 3, sblk2, dblk2)
        run_block(sblk, dblk)
        return carry

    lax.fori_loop(0, _NPAIR, pair_body, 0)
    pf_wait(sblk2, dblk2)

    start = ebase + _NFULL * _CH
    pltpu.sync_copy(src_hbm.at[pl.ds(start, _TAIL)], sidx_t)
    pltpu.sync_copy(dst_hbm.at[pl.ds(start, _TAIL)], didx_t)
    pltpu.async_copy(xtan_hbm.at[sidx_t], rows_t, sem).wait()
    pltpu.sync_copy(rows_t, acc.at[didx_t], add=True)

    plsc.subcore_barrier()

    pltpu.sync_copy(acc.at[pl.ds(rowbase, _RPT)],
                    out_hbm.at[pl.ds(c * _N + rowbase, _RPT)])

    @pl.when(s == _NS - 1)
    def _():
        pltpu.sync_copy(acc.at[pl.ds(_NS * _RPT, _RREM)],
                        out_hbm.at[pl.ds(c * _N + _NS * _RPT, _RREM)])


def _sc_scatter(x_tan, src, dst, zero_rows):
    mesh = plsc.VectorSubcoreMesh(core_axis_name="c", subcore_axis_name="s")
    f = pl.kernel(
        _sc_body,
        out_type=jax.ShapeDtypeStruct((_NC * _N, _F), jnp.float32),
        mesh=mesh,
        scratch_types=[
            pltpu.VMEM((_BE,), jnp.int32),
            pltpu.VMEM((_BE,), jnp.int32),
            pltpu.VMEM((_BE,), jnp.int32),
            pltpu.VMEM((_BE,), jnp.int32),
            pltpu.VMEM((_CH, _F), jnp.float32),
            pltpu.VMEM((_CH, _F), jnp.float32),
            pltpu.VMEM((_TAIL,), jnp.int32),
            pltpu.VMEM((_TAIL,), jnp.int32),
            pltpu.VMEM((_TAIL, _F), jnp.float32),
            pltpu.VMEM_SHARED((_N, _F), jnp.float32),
            pltpu.SemaphoreType.DMA,
            pltpu.SemaphoreType.DMA,
            pltpu.SemaphoreType.DMA,
            pltpu.SemaphoreType.DMA,
        ],
    )
    return f(x_tan, src, dst, zero_rows)


_B2 = 2000
_NB2 = _N // _B2


_FH = _F - 1


def _tail_body(xt_ref, p0_ref, p1_ref, b_ref,
               w0_ref, b0_ref, w1_ref, b1_ref, wc_ref, bc_ref,
               olog_ref, oprob_ref, acc_ref):
    i = pl.program_id(0)
    dn = (((1,), (1,)), ((), ()))

    htan = xt_ref[...] + p0_ref[...] + p1_ref[...]
    s1 = _pair_sigma(jnp.sum(htan * htan, axis=1, keepdims=True), 1.0, 2.0)
    mx = s1 * lax.dot_general(htan, w0_ref[...], dn,
                              preferred_element_type=jnp.float32) \
        + b0_ref[...]
    s2 = _pair_sigma(jnp.sum(mx * mx, axis=1, keepdims=True), 2.0, 2.0)
    a = jax.nn.relu(mx)
    ra = jnp.sum(a * a, axis=1, keepdims=True)
    s3 = _pair_sigma(s2 * s2 * ra, 2.0, 2.0)
    mx = (s2 * s3) * lax.dot_general(a, w1_ref[...], dn,
                                     preferred_element_type=jnp.float32) \
        + b1_ref[...]
    s4 = _pair_sigma(jnp.sum(mx * mx, axis=1, keepdims=True), 2.0, 2.0)
    a = jax.nn.relu(mx)
    ra = jnp.sum(a * a, axis=1, keepdims=True)
    s5 = _pair_sigma(s4 * s4 * ra, 1.0, 1.0)
    tt = (s4 * s5) * a

    bvals = b_ref[...].reshape(1, _B2)
    gid = lax.broadcasted_iota(jnp.int32, (_G, _B2), 0)
    oh = jnp.where(gid == bvals, 1.0, 0.0)
    pp = jnp.dot(oh, tt, preferred_element_type=jnp.float32)

    @pl.when(i == 0)
    def _():
        acc_ref[...] = pp

    @pl.when(i > 0)
    def _():
        acc_ref[...] = acc_ref[...] + pp

    @pl.when(i == _NB2 - 1)
    def _():
        hp = acc_ref[...]
        s6 = _pair_sigma(jnp.sum(hp * hp, axis=1, keepdims=True), 1.0, 1.0)
        mxc = s6 * lax.dot_general(hp, wc_ref[...], dn,
                                   preferred_element_type=jnp.float32) \
            + bc_ref[...]
        s7 = _pair_sigma(jnp.sum(mxc * mxc, axis=1, keepdims=True), 1.0, 1.0)
        lt = s7 * mxc
        olog_ref[...] = lt
        m = jnp.maximum(jnp.max(lt, axis=1, keepdims=True), 0.0)
        e = jnp.exp(lt - m)
        denom = jnp.sum(e, axis=1, keepdims=True) + jnp.exp(-m)
        st = e / denom
        se, _ = _exp_coef(jnp.sum(st * st, axis=1, keepdims=True), 1.0)
        oprob_ref[...] = se * st


def _tail_call(x_tan, p0, p1, batch3, w0, b0, w1, b1, wc, bc):
    blk = lambda i: (i, 0)
    fixed = lambda i: (0, 0)
    return pl.pallas_call(
        _tail_body,
        grid=(_NB2,),
        in_specs=[
            pl.BlockSpec((_B2, _F), blk),
            pl.BlockSpec((_B2, _F), blk),
            pl.BlockSpec((_B2, _F), lambda i: (i + _NB2, 0)),
            pl.BlockSpec((1, 1, _B2), lambda i: (i, 0, 0)),
            pl.BlockSpec((_FH, _F), fixed),
            pl.BlockSpec((1, _FH), fixed),
            pl.BlockSpec((_FH, _FH), fixed),
            pl.BlockSpec((1, _FH), fixed),
            pl.BlockSpec((_CLS, _FH), fixed),
            pl.BlockSpec((1, _CLS), fixed),
        ],
        out_specs=[
            pl.BlockSpec((_G, _CLS), fixed),
            pl.BlockSpec((_G, _CLS), fixed),
        ],
        out_shape=[
            jax.ShapeDtypeStruct((_G, _CLS), jnp.float32),
            jax.ShapeDtypeStruct((_G, _CLS), jnp.float32),
        ],
        scratch_shapes=[pltpu.VMEM((_G, _FH), jnp.float32)],
    )(x_tan, p0, p1, batch3, w0, b0, w1, b1, wc, bc)


def kernel(x, edge_index, batch, W0, b0, W1, b1, Wc, bc):
    xt = x[:, 1:]
    x_tan = _xtan(xt)

    src = edge_index[0]
    dst = edge_index[1]
    zero_rows = jnp.zeros((_RPT, _F), jnp.float32)
    parts = _sc_scatter(x_tan, src, dst, zero_rows)

    batch3 = batch.reshape(_NB2, 1, _B2)

    olog, oprob = _tail_call(x_tan, parts, parts, batch3,
                             W0, b0.reshape(1, _FH),
                             W1, b1.reshape(1, _FH),
                             Wc, bc.reshape(1, _CLS))
    return olog, oprob


# speedup vs baseline: 1.1041x; 1.0231x over previous
import functools

import jax
import jax.numpy as jnp
from jax import lax
from jax.experimental import pallas as pl
from jax.experimental.pallas import tpu as pltpu
from jax.experimental.pallas import tpu_sc as plsc

_N = 10000
_E = 320000
_F = 128
_G = 64
_CLS = 39
_EPS = 1e-6
_MAXN = 1000.0

_NC = 2
_NS = 16
_EPT = _E // (_NC * _NS)
_CH = 128
_NFULL = _EPT // _CH
_TAIL = _EPT - _NFULL * _CH
_BCH = 26
_NBLK = _NFULL // _BCH
_BE = _BCH * _CH
_NPAIR = (_NBLK - 1) // 2
_RPT = 624
_RREM = _N - _NS * _RPT


def _sinh(a):
    return 0.5 * (jnp.exp(a) - jnp.exp(-a))


def _exp_coef(r2, sqrt_c):
    lnorm = jnp.sqrt(jnp.clip(r2 + _EPS, 1e-6, None))
    cut = jnp.minimum(lnorm, 50.0)
    coef = sqrt_c * _sinh(cut / sqrt_c) / lnorm
    m2 = coef * coef * r2
    norm = jnp.sqrt(jnp.clip(m2, 1e-12, None))
    scale = jnp.minimum(1.0, _MAXN / norm)
    sigma = coef * scale
    m2s = jnp.where(m2 == jnp.inf, 0.0, m2 * scale * scale)
    return sigma, m2s


def _log_coef(m2, sqrt_c1, sqrt_c2):
    head = jnp.sqrt(sqrt_c1 * sqrt_c1 + m2)
    z = jnp.clip(head / sqrt_c2 + _EPS, 1.0, None)
    dist = sqrt_c2 * jnp.log(z + jnp.sqrt(jnp.clip(z * z - 1.0, 1e-12,
                                                   None)))
    tmp = jnp.sqrt(jnp.clip(m2, 1e-12, None) + _EPS)
    return dist / tmp


def _pair_sigma(r2, sqrt_c1, sqrt_c2):
    sigma_e, m2 = _exp_coef(r2, sqrt_c1)
    return sigma_e * _log_coef(m2, sqrt_c1, sqrt_c2)


def _xtan_body(x_ref, o_ref):
    t = x_ref[...]
    r2 = jnp.sum(t * t, axis=1, keepdims=True)
    norm = jnp.sqrt(jnp.clip(r2, 1e-12, None))
    scale = jnp.minimum(1.0, _MAXN / norm)
    m2 = r2 * scale * scale
    o_ref[...] = (scale * _log_coef(m2, 1.0, 1.0)) * t


def _xtan(xt):
    b = 2000
    return pl.pallas_call(
        _xtan_body,
        grid=(_N // b,),
        in_specs=[pl.BlockSpec((b, _F), lambda i: (i, 0))],
        out_specs=pl.BlockSpec((b, _F), lambda i: (i, 0)),
        out_shape=jax.ShapeDtypeStruct((_N, _F), jnp.float32),
    )(xt)


def _sc_body(xtan_hbm, src_hbm, dst_hbm, zero_hbm, out_hbm,
             sblk, dblk, sblk2, dblk2, rows0, rows1, sidx_t, didx_t,
             rows_t, acc, sem, gsem, ssem, isem):
    c = lax.axis_index("c")
    s = lax.axis_index("s")
    tid = c * _NS + s

    rowbase = s * _RPT
    pltpu.sync_copy(zero_hbm, acc.at[pl.ds(rowbase, _RPT)])

    @pl.when(s == _NS - 1)
    def _():
        pltpu.sync_copy(zero_hbm.at[pl.ds(0, _RREM)],
                        acc.at[pl.ds(_NS * _RPT, _RREM)])

    plsc.subcore_barrier()

    ebase = tid * _EPT
    rows = (rows0, rows1)

    def run_block(sb, db):
        g = [None] * _BCH
        s = [None] * _BCH
        g[0] = pltpu.async_copy(xtan_hbm.at[sb.at[pl.ds(0, _CH)]],
                                rows0, gsem)
        for k in range(1, _BCH):
            cur = rows[(k - 1) % 2]
            nxt = rows[k % 2]
            if k >= 2:
                s[k - 2].wait()
            g[k] = pltpu.async_copy(
                xtan_hbm.at[sb.at[pl.ds(k * _CH, _CH)]], nxt, gsem)
            g[k - 1].wait()
            s[k - 1] = pltpu.async_copy(
                cur, acc.at[db.at[pl.ds((k - 1) * _CH, _CH)]],
                ssem, add=True)
        last = rows[(_BCH - 1) % 2]
        g[_BCH - 1].wait()
        s[_BCH - 1] = pltpu.async_copy(
            last, acc.at[db.at[pl.ds((_BCH - 1) * _CH, _CH)]],
            ssem, add=True)
        s[_BCH - 2].wait()
        s[_BCH - 1].wait()

    def pf(blk, sb, db):
        base = ebase + jnp.minimum(blk, _NBLK - 1) * _BE
        pltpu.async_copy(src_hbm.at[pl.ds(base, _BE)], sb, isem)
        pltpu.async_copy(dst_hbm.at[pl.ds(base, _BE)], db, isem)

    def pf_wait(sb, db):
        pltpu.make_async_copy(src_hbm.at[pl.ds(ebase, _BE)], sb, isem).wait()
        pltpu.make_async_copy(dst_hbm.at[pl.ds(ebase, _BE)], db, isem).wait()

    pltpu.sync_copy(src_hbm.at[pl.ds(ebase, _BE)], sblk)
    pltpu.sync_copy(dst_hbm.at[pl.ds(ebase, _BE)], dblk)
    pf(1, sblk2, dblk2)
    run_block(sblk, dblk)

    def pair_body(i, carry):
        pf_wait(sblk2, dblk2)
        pf(2 * i + 2, sblk, dblk)
        run_block(sblk2, dblk2)
        pf_wait(sblk, dblk)
        pf(2 * i + 3, sblk2, dblk2)
        run_block(sblk, dblk)
        return carry

    lax.fori_loop(0, _NPAIR, pair_body, 0)
    pf_wait(sblk2, dblk2)

    start = ebase + _NFULL * _CH
    pltpu.sync_copy(src_hbm.at[pl.ds(start, _TAIL)], sidx_t)
    pltpu.sync_copy(dst_hbm.at[pl.ds(start, _TAIL)], didx_t)
    pltpu.async_copy(xtan_hbm.at[sidx_t], rows_t, sem).wait()
    pltpu.sync_copy(rows_t, acc.at[didx_t], add=True)

    plsc.subcore_barrier()

    pltpu.sync_copy(acc.at[pl.ds(rowbase, _RPT)],
                    out_hbm.at[pl.ds(c * _N + rowbase, _RPT)])

    @pl.when(s == _NS - 1)
    def _():
        pltpu.sync_copy(acc.at[pl.ds(_NS * _RPT, _RREM)],
                        out_hbm.at[pl.ds(c * _N + _NS * _RPT, _RREM)])


def _sc_scatter(x_tan, src, dst, zero_rows):
    mesh = plsc.VectorSubcoreMesh(core_axis_name="c", subcore_axis_name="s")
    f = pl.kernel(
        _sc_body,
        out_type=jax.ShapeDtypeStruct((_NC * _N, _F), jnp.float32),
        mesh=mesh,
        scratch_types=[
            pltpu.VMEM((_BE,), jnp.int32),
            pltpu.VMEM((_BE,), jnp.int32),
            pltpu.VMEM((_BE,), jnp.int32),
            pltpu.VMEM((_BE,), jnp.int32),
            pltpu.VMEM((_CH, _F), jnp.float32),
            pltpu.VMEM((_CH, _F), jnp.float32),
            pltpu.VMEM((_TAIL,), jnp.int32),
            pltpu.VMEM((_TAIL,), jnp.int32),
            pltpu.VMEM((_TAIL, _F), jnp.float32),
            pltpu.VMEM_SHARED((_N, _F), jnp.float32),
            pltpu.SemaphoreType.DMA,
            pltpu.SemaphoreType.DMA,
            pltpu.SemaphoreType.DMA,
            pltpu.SemaphoreType.DMA,
        ],
    )
    return f(x_tan, src, dst, zero_rows)


_B2 = 2000
_NB2 = _N // _B2


_FH = _F - 1


def _tail_body(xt_ref, p0_ref, p1_ref, b_ref,
               w0_ref, b0_ref, w1_ref, b1_ref, wc_ref, bc_ref,
               olog_ref, oprob_ref, acc_ref):
    i = pl.program_id(0)
    dn = (((1,), (1,)), ((), ()))

    htan = xt_ref[...] + p0_ref[...] + p1_ref[...]
    s1 = _pair_sigma(jnp.sum(htan * htan, axis=1, keepdims=True), 1.0, 2.0)
    mx = s1 * lax.dot_general(htan, w0_ref[...], dn,
                              preferred_element_type=jnp.float32) \
        + b0_ref[...]
    s2 = _pair_sigma(jnp.sum(mx * mx, axis=1, keepdims=True), 2.0, 2.0)
    a = jax.nn.relu(mx)
    ra = jnp.sum(a * a, axis=1, keepdims=True)
    s3 = _pair_sigma(s2 * s2 * ra, 2.0, 2.0)
    mx = (s2 * s3) * lax.dot_general(a, w1_ref[...], dn,
                                     preferred_element_type=jnp.float32) \
        + b1_ref[...]
    s4 = _pair_sigma(jnp.sum(mx * mx, axis=1, keepdims=True), 2.0, 2.0)
    a = jax.nn.relu(mx)
    ra = jnp.sum(a * a, axis=1, keepdims=True)
    s5 = _pair_sigma(s4 * s4 * ra, 1.0, 1.0)
    tt = (s4 * s5) * a

    bvals = b_ref[...].reshape(1, _B2)
    gid = lax.broadcasted_iota(jnp.int32, (_G, _B2), 0)
    oh = jnp.where(gid == bvals, 1.0, 0.0)
    pp = jnp.dot(oh, tt, preferred_element_type=jnp.float32)

    @pl.when(i == 0)
    def _():
        acc_ref[...] = pp

    @pl.when(i > 0)
    def _():
        acc_ref[...] = acc_ref[...] + pp

    @pl.when(i == _NB2 - 1)
    def _():
        hp = acc_ref[...]
        s6 = _pair_sigma(jnp.sum(hp * hp, axis=1, keepdims=True), 1.0, 1.0)
        mxc = s6 * lax.dot_general(hp, wc_ref[...], dn,
                                   preferred_element_type=jnp.float32) \
            + bc_ref[...]
        s7 = _pair_sigma(jnp.sum(mxc * mxc, axis=1, keepdims=True), 1.0, 1.0)
        lt = s7 * mxc
        olog_ref[...] = lt
        m = jnp.maximum(jnp.max(lt, axis=1, keepdims=True), 0.0)
        e = jnp.exp(lt - m)
        denom = jnp.sum(e, axis=1, keepdims=True) + jnp.exp(-m)
        st = e / denom
        se, _ = _exp_coef(jnp.sum(st * st, axis=1, keepdims=True), 1.0)
        oprob_ref[...] = se * st


def _tail_call(x_tan, p0, p1, batch3, w0, b0, w1, b1, wc, bc):
    blk = lambda i: (i, 0)
    fixed = lambda i: (0, 0)
    return pl.pallas_call(
        _tail_body,
        grid=(_NB2,),
        in_specs=[
            pl.BlockSpec((_B2, _F), blk),
            pl.BlockSpec((_B2, _F), blk),
            pl.BlockSpec((_B2, _F), lambda i: (i + _NB2, 0)),
            pl.BlockSpec((1, 1, _B2), lambda i: (i, 0, 0)),
            pl.BlockSpec((_FH, _F), fixed),
            pl.BlockSpec((1, _FH), fixed),
            pl.BlockSpec((_FH, _FH), fixed),
            pl.BlockSpec((1, _FH), fixed),
            pl.BlockSpec((_CLS, _FH), fixed),
            pl.BlockSpec((1, _CLS), fixed),
        ],
        out_specs=[
            pl.BlockSpec((_G, _CLS), fixed),
            pl.BlockSpec((_G, _CLS), fixed),
        ],
        out_shape=[
            jax.ShapeDtypeStruct((_G, _CLS), jnp.float32),
            jax.ShapeDtypeStruct((_G, _CLS), jnp.float32),
        ],
        scratch_shapes=[pltpu.VMEM((_G, _FH), jnp.float32)],
    )(x_tan, p0, p1, batch3, w0, b0, w1, b1, wc, bc)


def kernel(x, edge_index, batch, W0, b0, W1, b1, Wc, bc):
    xt = x[:, 1:]
    x_tan = _xtan(xt)

    src = edge_index[0]
    dst = edge_index[1]
    zero_rows = jnp.zeros((_RPT, _F), jnp.float32)
    parts = _sc_scatter(x_tan, src, dst, zero_rows)

    batch3 = batch.reshape(_NB2, 1, _B2)

    olog, oprob = _tail_call(x_tan, parts, parts, batch3,
                             W0, b0.reshape(1, _FH),
                             W1, b1.reshape(1, _FH),
                             Wc, bc.reshape(1, _CLS))
    return olog, oprob
